# baseline (speedup 1.0000x reference)
import functools

import jax
import jax.numpy as jnp
from jax import lax
from jax.experimental import pallas as pl
from jax.experimental.pallas import tpu as pltpu
from jax.experimental.pallas import tpu_sc as plsc

_UNROLL = 8


@functools.lru_cache(maxsize=None)
def _build_gather(B, V, D):
    info = plsc.get_sparse_core_info()
    nc, ns, nl = info.num_cores, info.num_subcores, info.num_lanes
    nw = nc * ns
    d_per_w = D // nw
    bh = B // 2
    va = (V // 2 + 127) // 128 * 128
    vb = V - va
    step = nl * _UNROLL
    mesh = plsc.VectorSubcoreMesh(core_axis_name="c", subcore_axis_name="s")

    @functools.partial(
        pl.kernel,
        mesh=mesh,
        out_type=jax.ShapeDtypeStruct((D, B), jnp.float32),
        scratch_types=[
            pltpu.VMEM((va,), jnp.float32),
            pltpu.VMEM((vb,), jnp.float32),
            pltpu.VMEM((bh,), jnp.int32),
            pltpu.VMEM((2, bh), jnp.float32),
            pltpu.SemaphoreType.DMA,
            pltpu.SemaphoreType.DMA,
            pltpu.SemaphoreType.DMA,
        ],
        compiler_params=pltpu.CompilerParams(needs_layout_passes=False),
    )
    def gather(idx_hbm, table_hbm, out_hbm, rowa_v, rowb_v, idx_v, obuf_v,
               asem, bsem, wsem):
        wid = lax.axis_index("s") * nc + lax.axis_index("c")
        d0 = wid * d_per_w

        def stage_row(d):
            pltpu.async_copy(table_hbm.at[d, pl.ds(0, va)], rowa_v, asem)
            pltpu.async_copy(table_hbm.at[d, pl.ds(va, vb)], rowb_v, bsem)

        def gather_pass(row_v, split_lo, h, slot):
            def body(k, _):
                base = k * step
                for u in range(_UNROLL):
                    o = base + u * nl
                    iv = idx_v[pl.ds(o, nl)]
                    if split_lo:
                        a = jnp.minimum(iv, va - 1)
                        vals = plsc.load_gather(row_v, [a])
                        res = jnp.where(iv < va, vals, 0.0)
                    else:
                        a = jnp.maximum(iv - va, 0)
                        vals = plsc.load_gather(row_v, [a])
                        prev = obuf_v[slot, pl.ds(o, nl)]
                        res = jnp.where(iv >= va, vals, prev)
                    obuf_v[slot, pl.ds(o, nl)] = res
                return _

            lax.fori_loop(0, bh // step, body, 0)

        stage_row(d0)
        pending = []
        it = 0
        for dd in range(d_per_w):
            d = d0 + dd
            for h in range(2):
                slot = it % 2
                if it >= 2:
                    pltpu.make_async_copy(*pending[it - 2]).wait()
                pltpu.sync_copy(idx_hbm.at[pl.ds(h * bh, bh)], idx_v)
                if h == 0:
                    pltpu.make_async_copy(
                        table_hbm.at[d, pl.ds(0, va)], rowa_v, asem).wait()
                gather_pass(rowa_v, True, h, slot)
                if h == 0:
                    pltpu.make_async_copy(
                        table_hbm.at[d, pl.ds(va, vb)], rowb_v, bsem).wait()
                if h == 1 and dd + 1 < d_per_w:
                    pltpu.async_copy(
                        table_hbm.at[d + 1, pl.ds(0, va)], rowa_v, asem)
                gather_pass(rowb_v, False, h, slot)
                if h == 1 and dd + 1 < d_per_w:
                    pltpu.async_copy(
                        table_hbm.at[d + 1, pl.ds(va, vb)], rowb_v, bsem)
                args = (obuf_v.at[slot], out_hbm.at[d, pl.ds(h * bh, bh)],
                        wsem)
                pltpu.async_copy(*args)
                pending.append(args)
                it += 1
        for t in range(max(0, it - 2), it):
            pltpu.make_async_copy(*pending[t]).wait()

    def run(inputs, embeddings):
        p = jnp.transpose(embeddings)
        out_t = gather(inputs.astype(jnp.int32), p)
        return jnp.transpose(out_t)[None]

    return run


def kernel(inputs, embeddings):
    B = inputs.shape[0]
    V, D = embeddings.shape
    return _build_gather(B, V, D)(inputs, embeddings)


# speedup vs baseline: 1.6725x; 1.6725x over previous
import functools

import jax
import jax.numpy as jnp
from jax import lax
from jax.experimental import pallas as pl
from jax.experimental.pallas import tpu as pltpu
from jax.experimental.pallas import tpu_sc as plsc

_UNROLL = 8
_NQ = 4


@functools.lru_cache(maxsize=None)
def _build_gather(B, V, D):
    info = plsc.get_sparse_core_info()
    nc, ns, nl = info.num_cores, info.num_subcores, info.num_lanes
    nw = nc * ns
    d_per_w = D // nw
    half = B // 2
    qv = (V // _NQ + 127) // 128 * 128
    qstarts = [q * qv for q in range(_NQ)]
    qlens = [qv] * (_NQ - 1) + [V - (_NQ - 1) * qv]
    qmask_lo = [q * qv for q in range(_NQ)]
    step = nl * _UNROLL
    mesh = plsc.VectorSubcoreMesh(core_axis_name="c", subcore_axis_name="s")

    @functools.partial(
        pl.kernel,
        mesh=mesh,
        out_type=jax.ShapeDtypeStruct((D, B), jnp.float32),
        scratch_types=[
            pltpu.VMEM((1, V), jnp.float32),
            pltpu.VMEM((half,), jnp.int32),
            pltpu.VMEM((half,), jnp.float32),
            pltpu.VMEM((half,), jnp.float32),
            pltpu.SemaphoreType.DMA,
            pltpu.SemaphoreType.DMA,
            pltpu.SemaphoreType.DMA,
            pltpu.SemaphoreType.DMA,
            pltpu.SemaphoreType.DMA,
        ],
        compiler_params=pltpu.CompilerParams(needs_layout_passes=False),
    )
    def gather(idx_hbm, table_hbm, out_hbm, row_v, idx_v, ob0_v, ob1_v,
               qs0, qs1, qs2, qs3, wsem):
        wid = lax.axis_index("s") * nc + lax.axis_index("c")
        d0 = wid * d_per_w
        qsems = [qs0, qs1, qs2, qs3]
        obufs = [ob0_v, ob1_v]

        def quarter_args(d, q):
            return (table_hbm.at[pl.ds(d, 1), pl.ds(qstarts[q], qlens[q])],
                    row_v.at[pl.ds(0, 1), pl.ds(qstarts[q], qlens[q])],
                    qsems[q])

        def gather_pass(q, obuf, init):
            lo = qmask_lo[q]
            hi = qmask_lo[q + 1] if q + 1 < _NQ else V

            zero = jnp.zeros((nl,), jnp.int32)

            def body(k, _):
                base = k * step
                ivs = [
                    idx_v[pl.ds(base + u * nl, nl)]
                    for u in range(_UNROLL)
                ]
                vals = [plsc.load_gather(row_v, [zero, iv]) for iv in ivs]
                for u in range(_UNROLL):
                    iv, val = ivs[u], vals[u]
                    msk = (iv >= lo) & (iv < hi) if lo else iv < hi
                    res = jnp.where(msk, val, 0.0)
                    sl = pl.ds(base + u * nl, nl)
                    if init:
                        obuf[sl] = res
                    else:
                        obuf[sl] += res
                return _

            lax.fori_loop(0, half // step, body, 0)

        for q in range(_NQ):
            pltpu.async_copy(*quarter_args(d0, q))
        pending = []
        it = 0
        for dd in range(d_per_w):
            d = d0 + dd
            for h in range(2):
                obuf = obufs[h]
                if it >= 2:
                    pltpu.make_async_copy(*pending[it - 2]).wait()
                pltpu.sync_copy(idx_hbm.at[pl.ds(h * half, half)], idx_v)
                for q in range(_NQ):
                    if h == 0:
                        pltpu.make_async_copy(*quarter_args(d, q)).wait()
                    gather_pass(q, obuf, init=(q == 0))
                    if h == 1 and dd + 1 < d_per_w:
                        pltpu.async_copy(*quarter_args(d + 1, q))
                args = (obuf, out_hbm.at[d, pl.ds(h * half, half)], wsem)
                pltpu.async_copy(*args)
                pending.append(args)
                it += 1
        for t in range(max(0, it - 2), it):
            pltpu.make_async_copy(*pending[t]).wait()

    def run(inputs, embeddings):
        p = jnp.transpose(embeddings)
        out_t = gather(inputs.astype(jnp.int32), p)
        return jnp.transpose(out_t)[None]

    return run


def kernel(inputs, embeddings):
    B = inputs.shape[0]
    V, D = embeddings.shape
    return _build_gather(B, V, D)(inputs, embeddings)


# speedup vs baseline: 1.6935x; 1.0126x over previous
import functools

import jax
import jax.numpy as jnp
from jax import lax
from jax.experimental import pallas as pl
from jax.experimental.pallas import tpu as pltpu
from jax.experimental.pallas import tpu_sc as plsc

_UNROLL = 8
_NQ = 4


@functools.lru_cache(maxsize=None)
def _build_gather(B, V, D):
    info = plsc.get_sparse_core_info()
    nc, ns, nl = info.num_cores, info.num_subcores, info.num_lanes
    nw = nc * ns
    d_per_w = D // nw
    half = B // 2
    qv = (V // _NQ + 127) // 128 * 128
    qstarts = [q * qv for q in range(_NQ)]
    qlens = [qv] * (_NQ - 1) + [V - (_NQ - 1) * qv]
    qmask_lo = [q * qv for q in range(_NQ)]
    step = nl * _UNROLL
    mesh = plsc.VectorSubcoreMesh(core_axis_name="c", subcore_axis_name="s")

    @functools.partial(
        pl.kernel,
        mesh=mesh,
        out_type=jax.ShapeDtypeStruct((D, B), jnp.float32),
        scratch_types=[
            pltpu.VMEM((1, V), jnp.float32),
            pltpu.VMEM((half,), jnp.int32),
            pltpu.VMEM((half,), jnp.float32),
            pltpu.VMEM((half,), jnp.float32),
            pltpu.SemaphoreType.DMA,
            pltpu.SemaphoreType.DMA,
            pltpu.SemaphoreType.DMA,
            pltpu.SemaphoreType.DMA,
            pltpu.SemaphoreType.DMA,
        ],
        compiler_params=pltpu.CompilerParams(needs_layout_passes=False),
    )
    def gather(idx_hbm, table_hbm, out_hbm, row_v, idx_v, ob0_v, ob1_v,
               qs0, qs1, qs2, qs3, wsem):
        wid = lax.axis_index("s") * nc + lax.axis_index("c")
        d0 = wid * d_per_w
        qsems = [qs0, qs1, qs2, qs3]
        obufs = [ob0_v, ob1_v]

        def quarter_args(d, q):
            return (table_hbm.at[pl.ds(d, 1), pl.ds(qstarts[q], qlens[q])],
                    row_v.at[pl.ds(0, 1), pl.ds(qstarts[q], qlens[q])],
                    qsems[q])

        def gather_pass(q, obuf, init):
            lo = qmask_lo[q]
            hi = qmask_lo[q + 1] if q + 1 < _NQ else V

            zero = jnp.zeros((nl,), jnp.int32)

            def body(k, _):
                base = k * step
                ivs = [
                    idx_v[pl.ds(base + u * nl, nl)]
                    for u in range(_UNROLL)
                ]
                vals = [plsc.load_gather(row_v, [zero, iv]) for iv in ivs]
                for u in range(_UNROLL):
                    iv, val = ivs[u], vals[u]
                    msk = (iv >= lo) & (iv < hi) if lo else iv < hi
                    res = jnp.where(msk, val, 0.0)
                    sl = pl.ds(base + u * nl, nl)
                    if init:
                        obuf[sl] = res
                    else:
                        plsc.addupdate(obuf.at[sl], res)
                return _

            lax.fori_loop(0, half // step, body, 0)

        for q in range(_NQ):
            pltpu.async_copy(*quarter_args(d0, q))
        pending = []
        it = 0
        for dd in range(d_per_w):
            d = d0 + dd
            for h in range(2):
                obuf = obufs[h]
                if it >= 2:
                    pltpu.make_async_copy(*pending[it - 2]).wait()
                pltpu.sync_copy(idx_hbm.at[pl.ds(h * half, half)], idx_v)
                for q in range(_NQ):
                    if h == 0:
                        pltpu.make_async_copy(*quarter_args(d, q)).wait()
                    gather_pass(q, obuf, init=(q == 0))
                    if h == 1 and dd + 1 < d_per_w:
                        pltpu.async_copy(*quarter_args(d + 1, q))
                args = (obuf, out_hbm.at[d, pl.ds(h * half, half)], wsem)
                pltpu.async_copy(*args)
                pending.append(args)
                it += 1
        for t in range(max(0, it - 2), it):
            pltpu.make_async_copy(*pending[t]).wait()

    def run(inputs, embeddings):
        p = jnp.transpose(embeddings)
        out_t = gather(inputs.astype(jnp.int32), p)
        return jnp.transpose(out_t)[None]

    return run


def kernel(inputs, embeddings):
    B = inputs.shape[0]
    V, D = embeddings.shape
    return _build_gather(B, V, D)(inputs, embeddings)


# speedup vs baseline: 2.3269x; 1.3740x over previous
import functools

import jax
import jax.numpy as jnp
from jax import lax
from jax.experimental import pallas as pl
from jax.experimental.pallas import tpu as pltpu
from jax.experimental.pallas import tpu_sc as plsc

_UNROLL = 8
_NBQ = 4


@functools.lru_cache(maxsize=None)
def _build_gather(B, V, D):
    info = plsc.get_sparse_core_info()
    nc, ns, nl = info.num_cores, info.num_subcores, info.num_lanes
    nw = nc * ns
    d_per_w = D // nw
    half = B // 2
    bq = B // _NBQ
    step = nl * _UNROLL
    mesh = plsc.VectorSubcoreMesh(core_axis_name="c", subcore_axis_name="s")

    @functools.partial(
        pl.kernel,
        mesh=mesh,
        out_type=jax.ShapeDtypeStruct((D, B), jnp.float32),
        scratch_types=[
            pltpu.VMEM((V,), jnp.float32),
            pltpu.VMEM((half,), jnp.int32),
            pltpu.VMEM((half,), jnp.int32),
            pltpu.VMEM((1, bq), jnp.float32),
            pltpu.VMEM((1, bq), jnp.float32),
            pltpu.SemaphoreType.DMA,
            pltpu.SemaphoreType.DMA,
            pltpu.SemaphoreType.DMA,
        ],
        compiler_params=pltpu.CompilerParams(needs_layout_passes=False),
    )
    def gather(idx_hbm, table_hbm, out_hbm, row_v, idx0_v, idx1_v,
               ob0_v, ob1_v, rsem, isem, wsem):
        wid = lax.axis_index("s") * nc + lax.axis_index("c")
        d0 = wid * d_per_w
        idxs = [idx0_v, idx1_v]
        obufs = [ob0_v, ob1_v]

        pltpu.async_copy(table_hbm.at[d0], row_v, rsem)
        for h in range(2):
            pltpu.async_copy(idx_hbm.at[pl.ds(h * half, half)],
                             idxs[h], isem)
        for h in range(2):
            pltpu.make_async_copy(idx_hbm.at[pl.ds(h * half, half)],
                                  idxs[h], isem).wait()

        pending = []
        it = 0
        for dd in range(d_per_w):
            d = d0 + dd
            pltpu.make_async_copy(table_hbm.at[d], row_v, rsem).wait()
            for q in range(_NBQ):
                idx_v = idxs[(q * bq) // half]
                off = (q * bq) % half
                obuf = obufs[it % 2]
                if it >= 2:
                    pltpu.make_async_copy(*pending[it - 2]).wait()

                def body(k, _):
                    base = k * step
                    ivs = [
                        idx_v[pl.ds(off + base + u * nl, nl)]
                        for u in range(_UNROLL)
                    ]
                    vals = [plsc.load_gather(row_v, [iv]) for iv in ivs]
                    for u in range(_UNROLL):
                        obuf[0, pl.ds(base + u * nl, nl)] = vals[u]
                    return _

                lax.fori_loop(0, bq // step, body, 0)
                if q == _NBQ - 1 and dd + 1 < d_per_w:
                    pltpu.async_copy(table_hbm.at[d + 1], row_v, rsem)
                args = (obuf,
                        out_hbm.at[pl.ds(d, 1), pl.ds(q * bq, bq)], wsem)
                pltpu.async_copy(*args)
                pending.append(args)
                it += 1
        for t in range(max(0, it - 2), it):
            pltpu.make_async_copy(*pending[t]).wait()

    def run(inputs, embeddings):
        p = jnp.transpose(embeddings)
        out_t = gather(inputs.astype(jnp.int32), p)
        return jnp.transpose(out_t)[None]

    return run


def kernel(inputs, embeddings):
    B = inputs.shape[0]
    V, D = embeddings.shape
    return _build_gather(B, V, D)(inputs, embeddings)


# speedup vs baseline: 2.3336x; 1.0029x over previous
import functools

import jax
import jax.numpy as jnp
from jax import lax
from jax.experimental import pallas as pl
from jax.experimental.pallas import tpu as pltpu
from jax.experimental.pallas import tpu_sc as plsc

_UNROLL = 8
_NBQ = 4


@functools.lru_cache(maxsize=None)
def _build_gather(B, V, D):
    info = plsc.get_sparse_core_info()
    nc, ns, nl = info.num_cores, info.num_subcores, info.num_lanes
    nw = nc * ns
    d_per_w = D // nw
    half = B // 2
    bq = B // _NBQ
    step = nl * _UNROLL
    mesh = plsc.VectorSubcoreMesh(core_axis_name="c", subcore_axis_name="s")

    vh = (V // 2 + 127) // 128 * 128
    vparts = [(0, vh), (vh, V - vh)]

    @functools.partial(
        pl.kernel,
        mesh=mesh,
        out_type=jax.ShapeDtypeStruct((D, B), jnp.float32),
        scratch_types=[
            pltpu.VMEM((1, V), jnp.float32),
            pltpu.VMEM((half,), jnp.int32),
            pltpu.VMEM((half,), jnp.int32),
            pltpu.VMEM((1, bq), jnp.float32),
            pltpu.VMEM((1, bq), jnp.float32),
            pltpu.SemaphoreType.DMA,
            pltpu.SemaphoreType.DMA,
            pltpu.SemaphoreType.DMA,
            pltpu.SemaphoreType.DMA,
        ],
        compiler_params=pltpu.CompilerParams(needs_layout_passes=False),
    )
    def gather(idx_hbm, table_hbm, out_hbm, row_v, idx0_v, idx1_v,
               ob0_v, ob1_v, rsem0, rsem1, isem, wsem):
        wid = lax.axis_index("s") * nc + lax.axis_index("c")
        d0 = wid * d_per_w
        idxs = [idx0_v, idx1_v]
        obufs = [ob0_v, ob1_v]
        rsems = [rsem0, rsem1]
        zero = jnp.zeros((nl,), jnp.int32)

        def row_args(d, j):
            s, l = vparts[j]
            return (table_hbm.at[pl.ds(d, 1), pl.ds(s, l)],
                    row_v.at[pl.ds(0, 1), pl.ds(s, l)], rsems[j])

        def stage_row(d):
            pltpu.async_copy(*row_args(d, 0))
            pltpu.async_copy(*row_args(d, 1))

        def wait_row(d):
            pltpu.make_async_copy(*row_args(d, 0)).wait()
            pltpu.make_async_copy(*row_args(d, 1)).wait()

        stage_row(d0)
        for h in range(2):
            pltpu.async_copy(idx_hbm.at[pl.ds(h * half, half)],
                             idxs[h], isem)
        for h in range(2):
            pltpu.make_async_copy(idx_hbm.at[pl.ds(h * half, half)],
                                  idxs[h], isem).wait()

        pending = []
        it = 0
        for dd in range(d_per_w):
            d = d0 + dd
            wait_row(d)
            for q in range(_NBQ):
                idx_v = idxs[(q * bq) // half]
                off = (q * bq) % half
                obuf = obufs[it % 2]
                if it >= 2:
                    pltpu.make_async_copy(*pending[it - 2]).wait()

                def body(k, _):
                    base = k * step
                    ivs = [
                        idx_v[pl.ds(off + base + u * nl, nl)]
                        for u in range(_UNROLL)
                    ]
                    vals = [
                        plsc.load_gather(row_v, [zero, iv]) for iv in ivs
                    ]
                    for u in range(_UNROLL):
                        obuf[0, pl.ds(base + u * nl, nl)] = vals[u]
                    return _

                lax.fori_loop(0, bq // step, body, 0)
                if q == _NBQ - 1 and dd + 1 < d_per_w:
                    stage_row(d + 1)
                args = (obuf,
                        out_hbm.at[pl.ds(d, 1), pl.ds(q * bq, bq)], wsem)
                pltpu.async_copy(*args)
                pending.append(args)
                it += 1
        for t in range(max(0, it - 2), it):
            pltpu.make_async_copy(*pending[t]).wait()

    def run(inputs, embeddings):
        p = jnp.transpose(embeddings)
        out_t = gather(inputs.astype(jnp.int32), p)
        return jnp.transpose(out_t)[None]

    return run


def kernel(inputs, embeddings):
    B = inputs.shape[0]
    V, D = embeddings.shape
    return _build_gather(B, V, D)(inputs, embeddings)
